# baseline (speedup 1.0000x reference)
import functools

import jax
import jax.numpy as jnp
from jax import lax
from jax.experimental import pallas as pl
from jax.experimental.pallas import tpu as pltpu
from jax.experimental.pallas import tpu_sc as plsc

N_USERS = 4000
N_ITEMS = 6000
N = N_USERS + N_ITEMS
R = 3
D = 128
L = 3
E = 320000

NC = 2
NS = 16
NW = NC * NS
CHUNK = 128
NBLK = 2
NCHUNK = 40
NGRP = NCHUNK // 2
EPT = NBLK * NCHUNK * CHUNK
EPAD = NW * EPT
N_PAD = 10112
ROWS_PT = N_PAD // NS

BN = 400
NB = N // BN


def _leaky(x):
    return jnp.where(x >= 0, x, 0.01 * x)


def _dot(a, b):
    return lax.dot_general(a, b, (((1,), (0,)), ((), ())),
                           preferred_element_type=jnp.float32)


def _tc_rela(rel_emb, W_rel):
    def body(re_ref, w_ref, rt_ref, rm_ref):
        r = re_ref[...]
        acc = r
        rt_ref[0] = r
        for k in range(L):
            r = _dot(r, w_ref[k])
            rt_ref[k + 1] = r
            acc = acc + r
        rm_ref[...] = (acc * (1.0 / (L + 1)))[:, None, :]

    return pl.pallas_call(
        body,
        grid=(1,),
        in_specs=[pl.BlockSpec((R, D), lambda b: (0, 0)),
                  pl.BlockSpec((L, D, D), lambda b: (0, 0, 0))],
        out_specs=[pl.BlockSpec((L + 1, R, D), lambda b: (0, 0, 0)),
                   pl.BlockSpec((R, 1, D), lambda b: (0, 0, 0))],
        out_shape=[jax.ShapeDtypeStruct((L + 1, R, D), jnp.float32),
                   jax.ShapeDtypeStruct((R, 1, D), jnp.float32)],
    )(rel_emb, W_rel)


def _split_store(y_ref, i, y):
    y_ref[i] = y


def _tc_pre(base, rela, W):
    def body(b_ref, r_ref, w_ref, y_ref):
        x = b_ref[...]
        w = w_ref[...]
        for i in range(R):
            _split_store(y_ref, i, _dot(x * r_ref[i][None, :], w))

    return pl.pallas_call(
        body,
        grid=(NB,),
        in_specs=[pl.BlockSpec((BN, D), lambda b: (b, 0)),
                  pl.BlockSpec((R, D), lambda b: (0, 0)),
                  pl.BlockSpec((D, D), lambda b: (0, 0))],
        out_specs=pl.BlockSpec((R, BN, D), lambda b: (0, b, 0)),
        out_shape=jax.ShapeDtypeStruct((R, N, D), jnp.float32),
    )(base, rela, W)


def _tc_mid(part, acc, rela, W):
    def body(p_ref, a_ref, r_ref, w_ref, y_ref, ao_ref):
        w = w_ref[...]
        for i in range(R):
            e = _leaky(p_ref[0, i] + p_ref[1, i])
            ao_ref[i] = a_ref[i] + e
            _split_store(y_ref, i, _dot(e * r_ref[i][None, :], w))

    return pl.pallas_call(
        body,
        grid=(NB,),
        in_specs=[pl.BlockSpec((NC, R, BN, D), lambda b: (0, 0, b, 0)),
                  pl.BlockSpec((R, BN, D), lambda b: (0, b, 0)),
                  pl.BlockSpec((R, D), lambda b: (0, 0)),
                  pl.BlockSpec((D, D), lambda b: (0, 0))],
        out_specs=[pl.BlockSpec((R, BN, D), lambda b: (0, b, 0)),
                   pl.BlockSpec((R, BN, D), lambda b: (0, b, 0))],
        out_shape=[jax.ShapeDtypeStruct((R, N, D), jnp.float32),
                   jax.ShapeDtypeStruct((R, N, D), jnp.float32)],
    )(part, acc, rela, W)


def _tc_fin(part, acc, base):
    def body(p_ref, a_ref, b_ref, o_ref):
        x = b_ref[...]
        for i in range(R):
            e = _leaky(p_ref[0, i] + p_ref[1, i])
            o_ref[:, i, :] = (x + a_ref[i] + e) * 0.25

    return pl.pallas_call(
        body,
        grid=(NB,),
        in_specs=[pl.BlockSpec((NC, R, BN, D), lambda b: (0, 0, b, 0)),
                  pl.BlockSpec((R, BN, D), lambda b: (0, b, 0)),
                  pl.BlockSpec((BN, D), lambda b: (b, 0))],
        out_specs=pl.BlockSpec((BN, R, D), lambda b: (b, 0, 0)),
        out_shape=jax.ShapeDtypeStruct((N, R, D), jnp.float32),
    )(part, acc, base)


def _sc_spmm(yall, cols, rows, vals, zeros):
    mesh = plsc.VectorSubcoreMesh(core_axis_name="c", subcore_axis_name="s",
                                  num_cores=NC, num_subcores=NS)

    @functools.partial(
        pl.kernel,
        out_type=jax.ShapeDtypeStruct((NC, R, N_PAD, D), jnp.float32),
        mesh=mesh,
        scratch_types=[
            pltpu.VMEM_SHARED((N_PAD, D), jnp.float32),
            pltpu.VMEM((NCHUNK, CHUNK), jnp.int32),
            pltpu.VMEM((NCHUNK, CHUNK), jnp.int32),
            pltpu.VMEM((NCHUNK, CHUNK), jnp.float32),
            pltpu.VMEM((2, CHUNK, D), jnp.float32),
        ] + [pltpu.SemaphoreType.DMA] * 2,
    )
    def k(yr, colsr, rowsr, valsr, zerosr, partr,
          accum, colbuf, rowbuf, valbuf, gbuf, gsem0, gsem1):
        gsem = (gsem0, gsem1)
        c = lax.axis_index("c")
        s = lax.axis_index("s")
        w = c * NS + s
        rslice = pl.ds(pl.multiple_of(s * ROWS_PT, 8), ROWS_PT)

        def rel_body(i, _):
            pltpu.sync_copy(zerosr.at[rslice], accum.at[rslice])
            plsc.subcore_barrier()

            def blk_body(sb, _):
                pltpu.sync_copy(colsr.at[i, w, sb], colbuf)
                pltpu.sync_copy(rowsr.at[i, w, sb], rowbuf)
                pltpu.sync_copy(valsr.at[i, w, sb], valbuf)

                pltpu.async_copy(yr.at[colbuf.at[0]], gbuf.at[0], gsem[0])

                def pair_body(jj, _):
                    for h in range(2):
                        j = 2 * jj + h
                        oh = 1 - h
                        pltpu.make_async_copy(
                            yr.at[colbuf.at[j]], gbuf.at[h], gsem[h]).wait()
                        if h == 0:
                            pltpu.async_copy(yr.at[colbuf.at[j + 1]],
                                             gbuf.at[oh], gsem[oh])
                        else:
                            @pl.when(jj < NGRP - 1)
                            def _(j=j, oh=oh):
                                pltpu.async_copy(yr.at[colbuf.at[j + 1]],
                                                 gbuf.at[oh], gsem[oh])

                        def scale_body(t, _, h=h, j=j):
                            vv = valbuf[j, pl.ds(t * 16, 16)]
                            for u in range(16):
                                v = vv[u]
                                e = t * 16 + u
                                for q in range(D // 16):
                                    sl = pl.ds(q * 16, 16)
                                    gbuf[h, e, sl] = gbuf[h, e, sl] * v
                            return 0

                        lax.fori_loop(0, CHUNK // 16, scale_body, 0)
                        pltpu.sync_copy(gbuf.at[h],
                                        accum.at[rowbuf.at[j]], add=True)
                    return 0

                lax.fori_loop(0, NGRP, pair_body, 0)
                return 0

            lax.fori_loop(0, NBLK, blk_body, 0)
            plsc.subcore_barrier()
            pltpu.sync_copy(accum.at[rslice], partr.at[c, i, rslice])
            return 0

        lax.fori_loop(0, R, rel_body, 0)

    return k(yall, cols, rows, vals, zeros)


def kernel(adj_idx, adj_val, user_embedding, item_embedding,
           relation_embedding, W_gc, W_rel):
    base = jnp.concatenate([user_embedding, item_embedding], axis=0)
    cols = adj_idx[:, 1, :].astype(jnp.int32)
    rows = adj_idx[:, 0, :].astype(jnp.int32)
    pad = EPAD - E
    colsf = cols + (jnp.arange(R, dtype=jnp.int32) * N)[:, None]
    cols_p = jnp.pad(colsf, ((0, 0), (0, pad))).reshape(
        R, NW, NBLK, NCHUNK, CHUNK)
    rows_p = jnp.pad(rows, ((0, 0), (0, pad))).reshape(
        R, NW, NBLK, NCHUNK, CHUNK)
    vals_p = jnp.pad(adj_val, ((0, 0), (0, pad))).reshape(
        R, NW, NBLK, NCHUNK, CHUNK)
    zeros = jnp.zeros((N_PAD, D), jnp.float32)

    rt, rmean = _tc_rela(relation_embedding, W_rel)

    yall = _tc_pre(base, rt[0], W_gc[0])
    acc = jnp.zeros((R, N, D), jnp.float32)
    for k in range(1, L):
        part = _sc_spmm(yall.reshape(R * N, D),
                        cols_p, rows_p, vals_p, zeros)
        yall, acc = _tc_mid(part, acc, rt[k], W_gc[k])
    part = _sc_spmm(yall.reshape(R * N, D),
                    cols_p, rows_p, vals_p, zeros)
    all_emb = _tc_fin(part, acc, base)

    u_g = all_emb[:N_USERS]
    i_g = jnp.concatenate(
        [all_emb[N_USERS:], jnp.zeros((1, R, D), jnp.float32)], axis=0)
    return (u_g, i_g, rmean)


# speedup vs baseline: 1.2095x; 1.2095x over previous
import functools

import jax
import jax.numpy as jnp
from jax import lax
from jax.experimental import pallas as pl
from jax.experimental.pallas import tpu as pltpu
from jax.experimental.pallas import tpu_sc as plsc

N_USERS = 4000
N_ITEMS = 6000
N = N_USERS + N_ITEMS
R = 3
D = 128
L = 3
E = 320000

NC = 2
NS = 16
NW = NC * NS
CHUNK = 128
EPW = -(-E // (NW * CHUNK)) * CHUNK
NCHUNK = EPW // CHUNK
EPAD = EPW * NW
N_PAD = 10240
ROWS_PT = N_PAD // NS

BN = 400
NB = N // BN


def _leaky(x):
    return jnp.where(x >= 0, x, 0.01 * x)


def _dot(a, b):
    return lax.dot_general(a, b, (((1,), (0,)), ((), ())),
                           preferred_element_type=jnp.float32)


def _tc_rela(rel_emb, W_rel):
    def body(re_ref, w_ref, rt_ref, rm_ref):
        r = re_ref[...]
        acc = r
        rt_ref[0] = r
        for k in range(L):
            r = _dot(r, w_ref[k])
            rt_ref[k + 1] = r
            acc = acc + r
        rm_ref[...] = (acc * (1.0 / (L + 1)))[:, None, :]

    return pl.pallas_call(
        body,
        grid=(1,),
        in_specs=[pl.BlockSpec((R, D), lambda b: (0, 0)),
                  pl.BlockSpec((L, D, D), lambda b: (0, 0, 0))],
        out_specs=[pl.BlockSpec((L + 1, R, D), lambda b: (0, 0, 0)),
                   pl.BlockSpec((R, 1, D), lambda b: (0, 0, 0))],
        out_shape=[jax.ShapeDtypeStruct((L + 1, R, D), jnp.float32),
                   jax.ShapeDtypeStruct((R, 1, D), jnp.float32)],
    )(rel_emb, W_rel)


def _tc_pre(base, rela, W):
    def body(b_ref, r_ref, w_ref, y0_ref, y1_ref, y2_ref):
        x = b_ref[...]
        w = w_ref[...]
        outs = (y0_ref, y1_ref, y2_ref)
        for i in range(R):
            outs[i][...] = _dot(x * r_ref[i][None, :], w)

    return pl.pallas_call(
        body,
        grid=(NB,),
        in_specs=[pl.BlockSpec((BN, D), lambda b: (b, 0)),
                  pl.BlockSpec((R, D), lambda b: (0, 0)),
                  pl.BlockSpec((D, D), lambda b: (0, 0))],
        out_specs=[pl.BlockSpec((BN, D), lambda b: (b, 0))] * R,
        out_shape=[jax.ShapeDtypeStruct((N, D), jnp.float32)] * R,
    )(base, rela, W)


def _tc_mid(part, acc, rela, W):
    def body(p_ref, a_ref, r_ref, w_ref, y0_ref, y1_ref, y2_ref, ao_ref):
        w = w_ref[...]
        outs = (y0_ref, y1_ref, y2_ref)
        for i in range(R):
            e = _leaky(p_ref[0, i] + p_ref[1, i])
            ao_ref[i] = a_ref[i] + e
            outs[i][...] = _dot(e * r_ref[i][None, :], w)

    return pl.pallas_call(
        body,
        grid=(NB,),
        in_specs=[pl.BlockSpec((NC, R, BN, D), lambda b: (0, 0, b, 0)),
                  pl.BlockSpec((R, BN, D), lambda b: (0, b, 0)),
                  pl.BlockSpec((R, D), lambda b: (0, 0)),
                  pl.BlockSpec((D, D), lambda b: (0, 0))],
        out_specs=[pl.BlockSpec((BN, D), lambda b: (b, 0))] * R
        + [pl.BlockSpec((R, BN, D), lambda b: (0, b, 0))],
        out_shape=[jax.ShapeDtypeStruct((N, D), jnp.float32)] * R
        + [jax.ShapeDtypeStruct((R, N, D), jnp.float32)],
    )(part, acc, rela, W)


def _tc_fin(part, acc, base):
    def body(p_ref, a_ref, b_ref, o_ref):
        x = b_ref[...]
        for i in range(R):
            e = _leaky(p_ref[0, i] + p_ref[1, i])
            o_ref[:, i, :] = (x + a_ref[i] + e) * 0.25

    return pl.pallas_call(
        body,
        grid=(NB,),
        in_specs=[pl.BlockSpec((NC, R, BN, D), lambda b: (0, 0, b, 0)),
                  pl.BlockSpec((R, BN, D), lambda b: (0, b, 0)),
                  pl.BlockSpec((BN, D), lambda b: (b, 0))],
        out_specs=pl.BlockSpec((BN, R, D), lambda b: (b, 0, 0)),
        out_shape=jax.ShapeDtypeStruct((N, R, D), jnp.float32),
    )(part, acc, base)


def _sc_spmm(y0, y1, y2, cols, rows, vals, zeros):
    mesh = plsc.VectorSubcoreMesh(core_axis_name="c", subcore_axis_name="s",
                                  num_cores=NC, num_subcores=NS)

    @functools.partial(
        pl.kernel,
        out_type=jax.ShapeDtypeStruct((NC, R, N_PAD, D), jnp.float32),
        mesh=mesh,
        scratch_types=[
            pltpu.VMEM_SHARED((N_PAD, D), jnp.float32),
            pltpu.VMEM((NCHUNK, CHUNK), jnp.int32),
            pltpu.VMEM((NCHUNK, CHUNK), jnp.int32),
            pltpu.VMEM((NCHUNK, CHUNK), jnp.float32),
            pltpu.VMEM((CHUNK, D), jnp.float32),
            pltpu.SemaphoreType.DMA,
        ],
    )
    def k(y0r, y1r, y2r, colsr, rowsr, valsr, zerosr, partr,
          accum, colbuf, rowbuf, valbuf, gbuf, gsem):
        c = lax.axis_index("c")
        s = lax.axis_index("s")
        w = c * NS + s
        ys = (y0r, y1r, y2r)
        rslice = pl.ds(pl.multiple_of(s * ROWS_PT, 8), ROWS_PT)
        for i in range(R):
            pltpu.sync_copy(zerosr.at[rslice], accum.at[rslice])
            pltpu.sync_copy(colsr.at[i, w], colbuf)
            pltpu.sync_copy(rowsr.at[i, w], rowbuf)
            pltpu.sync_copy(valsr.at[i, w], valbuf)
            plsc.subcore_barrier()

            yr = ys[i]

            def chunk_body(j, _):
                pltpu.async_copy(yr.at[colbuf.at[j]], gbuf, gsem).wait()

                def scale_body(t, _):
                    vv = valbuf[j, pl.ds(t * 16, 16)]
                    for u in range(16):
                        v = vv[u]
                        e = t * 16 + u
                        for q in range(D // 16):
                            sl = pl.ds(q * 16, 16)
                            gbuf[e, sl] = gbuf[e, sl] * v
                    return 0

                lax.fori_loop(0, CHUNK // 16, scale_body, 0)
                pltpu.sync_copy(gbuf, accum.at[rowbuf.at[j]], add=True)
                return 0

            lax.fori_loop(0, NCHUNK, chunk_body, 0)
            plsc.subcore_barrier()
            pltpu.sync_copy(accum.at[rslice], partr.at[c, i, rslice])

    return k(y0, y1, y2, cols, rows, vals, zeros)


def kernel(adj_idx, adj_val, user_embedding, item_embedding,
           relation_embedding, W_gc, W_rel):
    base = jnp.concatenate([user_embedding, item_embedding], axis=0)
    cols = adj_idx[:, 1, :].astype(jnp.int32)
    rows = adj_idx[:, 0, :].astype(jnp.int32)
    pad = EPAD - E
    cols_p = jnp.pad(cols, ((0, 0), (0, pad))).reshape(R, NW, NCHUNK, CHUNK)
    rows_p = jnp.pad(rows, ((0, 0), (0, pad))).reshape(R, NW, NCHUNK, CHUNK)
    vals_p = jnp.pad(adj_val, ((0, 0), (0, pad))).reshape(R, NW, NCHUNK, CHUNK)
    zeros = jnp.zeros((N_PAD, D), jnp.float32)

    rt, rmean = _tc_rela(relation_embedding, W_rel)

    y0, y1, y2 = _tc_pre(base, rt[0], W_gc[0])
    acc = jnp.zeros((R, N, D), jnp.float32)
    for k in range(1, L):
        part = _sc_spmm(y0, y1, y2, cols_p, rows_p, vals_p, zeros)
        y0, y1, y2, acc = _tc_mid(part, acc, rt[k], W_gc[k])
    part = _sc_spmm(y0, y1, y2, cols_p, rows_p, vals_p, zeros)
    all_emb = _tc_fin(part, acc, base)

    u_g = all_emb[:N_USERS]
    i_g = jnp.concatenate(
        [all_emb[N_USERS:], jnp.zeros((1, R, D), jnp.float32)], axis=0)
    return (u_g, i_g, rmean)
